# baseline (speedup 1.0000x reference)
import functools

import jax
import jax.numpy as jnp
from jax import lax
from jax.experimental import pallas as pl
from jax.experimental.pallas import tpu as pltpu
from jax.experimental.pallas import tpu_sc as plsc

_K = 512
_D = 32
_B = 16384
_G = 5
_BLK = 2048
_NB = _B // _BLK
_LOSS_SCALE = 1.25 / (_B * _D)

_NC = 2
_NS = 16
_NW = _NC * _NS
_TOT = _G * _B
_PER_W = _TOT // _NW
_CH = 128
_NCH = _PER_W // _CH


def _tc_body(x0, x1, x2, x3, x4, w0, w1, w2, w3, w4,
             idx_out, gidx_out, loss_out):
    b = pl.program_id(0)

    @pl.when(b == 0)
    def _init():
        loss_out[0, 0] = jnp.float32(0.0)

    total = jnp.float32(0.0)
    for g, (xr, wr) in enumerate(zip((x0, x1, x2, x3, x4),
                                     (w0, w1, w2, w3, w4))):
        x = xr[...]
        w = wr[...]
        x2 = jnp.sum(x * x, axis=1, keepdims=True)
        w2 = jnp.sum(w * w, axis=1)
        xw = lax.dot_general(x, w, (((1,), (1,)), ((), ())),
                             preferred_element_type=jnp.float32)
        dist = x2 + w2[None, :] - 2.0 * xw
        minv = jnp.min(dist, axis=1)
        iota = lax.broadcasted_iota(jnp.int32, dist.shape, 1)
        idx = jnp.min(jnp.where(dist == minv[:, None], iota, _K), axis=1)
        idx_out[g, :] = idx
        gidx_out[g, :] = idx + jnp.int32(g * _K)
        total = total + jnp.sum(minv)

    loss_out[0, 0] += total * jnp.float32(_LOSS_SCALE)


_tc_call = pl.pallas_call(
    _tc_body,
    grid=(_NB,),
    in_specs=[pl.BlockSpec((_BLK, _D), lambda b: (b, 0))] * _G
    + [pl.BlockSpec((_K, _D), lambda b: (0, 0))] * _G,
    out_specs=[
        pl.BlockSpec((_G, _BLK), lambda b: (0, b)),
        pl.BlockSpec((_G, _BLK), lambda b: (0, b)),
        pl.BlockSpec(memory_space=pltpu.SMEM, block_shape=(1, 1),
                     index_map=lambda b: (0, 0)),
    ],
    out_shape=[
        jax.ShapeDtypeStruct((_G, _B), jnp.int32),
        jax.ShapeDtypeStruct((_G, _B), jnp.int32),
        jax.ShapeDtypeStruct((1, 1), jnp.float32),
    ],
)


def _sc_gather_body(gidx_hbm, table_hbm, out_hbm, idx_v, rows_v, sem):
    wid = lax.axis_index("s") * _NC + lax.axis_index("c")
    base = wid * _PER_W
    pltpu.sync_copy(gidx_hbm.at[wid], idx_v)
    copies = []
    for j in range(_NCH):
        copies.append(
            pltpu.async_copy(table_hbm.at[idx_v.at[j]],
                             rows_v.at[pl.ds(j * _CH, _CH)], sem))
    for c in copies:
        c.wait()
    pltpu.sync_copy(rows_v, out_hbm.at[pl.ds(base, _PER_W)])


@functools.cache
def _sc_gather():
    return functools.partial(
        pl.kernel,
        out_type=jax.ShapeDtypeStruct((_TOT, _D), jnp.float32),
        mesh=plsc.VectorSubcoreMesh(core_axis_name="c", subcore_axis_name="s"),
        compiler_params=pltpu.CompilerParams(use_tc_tiling_on_sc=False),
        scratch_types=[
            pltpu.VMEM((_NCH, _CH), jnp.int32),
            pltpu.VMEM((_PER_W, _D), jnp.float32),
            pltpu.SemaphoreType.DMA,
        ],
    )(_sc_gather_body)


@jax.jit
def kernel(head_spine, left_arm, right_arm, left_leg, right_leg,
           W_head_spine, W_left_arm, W_right_arm, W_left_leg, W_right_leg):
    xs = (head_spine, left_arm, right_arm, left_leg, right_leg)
    ws = (W_head_spine, W_left_arm, W_right_arm, W_left_leg, W_right_leg)
    idx, gidx, loss = _tc_call(*xs, *ws)
    table = jnp.concatenate(ws, axis=0)
    gidx3 = gidx.reshape(_NW, _NCH, _CH)
    quant = _sc_gather()(gidx3, table)
    return quant.reshape(_G, _B, _D), idx, loss[0, 0]
